# baseline (speedup 1.0000x reference)
import functools

import jax
import jax.numpy as jnp
from jax import lax
from jax.experimental import pallas as pl
from jax.experimental.pallas import tpu as pltpu
from jax.experimental.pallas import tpu_sc as plsc

_NUM_CORES = 2
_NUM_SUBCORES = 16

_C = 128
_NB = 10
_LAG_G = 1
_LAG_S = 5
_NACC = 10240


def _mm_bias_split(x, W, b, BR):
  N, K = x.shape
  Do = W.shape[1]
  Dc = Do // 2
  G = N // BR

  def body(x_ref, w_ref, b_ref, o0_ref, o1_ref):
    r = jnp.dot(x_ref[...], w_ref[...],
                preferred_element_type=jnp.float32) + b_ref[...]
    o0_ref[...] = r[:, :Dc]
    o1_ref[...] = r[:, Dc:]

  return pl.pallas_call(
      body,
      grid=(G,),
      in_specs=[
          pl.BlockSpec((BR, K), lambda i: (i, 0)),
          pl.BlockSpec((K, Do), lambda i: (0, 0)),
          pl.BlockSpec((1, Do), lambda i: (0, 0)),
      ],
      out_specs=[
          pl.BlockSpec((BR, Dc), lambda i: (i, 0)),
          pl.BlockSpec((BR, Dc), lambda i: (i, 0)),
      ],
      out_shape=[
          jax.ShapeDtypeStruct((N, Dc), jnp.float32),
          jax.ShapeDtypeStruct((N, Dc), jnp.float32),
      ],
  )(x, W, b.reshape(1, Do))


def _aggregate(sup0, sup1, adj, n_out):
  Dc = sup0.shape[1]
  E = adj.shape[1]
  NCG = E // _C
  NCH = NCG // _NUM_SUBCORES
  NEXTRA = NCG - NCH * _NUM_SUBCORES
  RP = _NACC // _NUM_SUBCORES

  zeros = jnp.zeros((RP, Dc), jnp.float32)
  mesh = plsc.VectorSubcoreMesh(core_axis_name="c", subcore_axis_name="s")

  @functools.partial(
      pl.kernel,
      mesh=mesh,
      compiler_params=pltpu.CompilerParams(use_tc_tiling_on_sc=False),
      out_type=jax.ShapeDtypeStruct((n_out, 2 * Dc), jnp.float32),
      scratch_types=[
          pltpu.VMEM((_NB, 2, _C), jnp.int32),
          pltpu.VMEM((_NB, _C, Dc), jnp.float32),
          pltpu.VMEM_SHARED((_NACC, Dc), jnp.float32),
          pltpu.SemaphoreType.DMA((_NB,)),
          pltpu.SemaphoreType.DMA((_NB,)),
          pltpu.SemaphoreType.DMA((_NB,)),
      ],
  )
  def agg(sup0_h, sup1_h, adj_h, zero_h, out_h, idx_v, rows_v, acc,
          isem, gsem, ssem):
    cid = lax.axis_index("c")
    sid = lax.axis_index("s")
    r0 = sid * RP
    NCHT = NCH + jnp.where(sid < NEXTRA, 1, 0)
    pltpu.sync_copy(zero_h, acc.at[pl.ds(r0, RP)])
    plsc.subcore_barrier()

    def fire_idx(j, p):
      e0 = (sid + _NUM_SUBCORES * j) * _C
      pltpu.async_copy(adj_h.at[0, pl.ds(e0, _C)], idx_v.at[p, 0], isem.at[p])
      pltpu.async_copy(adj_h.at[1, pl.ds(e0, _C)], idx_v.at[p, 1], isem.at[p])

    def wait_idx(p):
      pltpu.make_async_copy(adj_h.at[pl.ds(0, 2), pl.ds(0, _C)],
                            idx_v.at[p], isem.at[p]).wait()

    def fire_gather(p, sup_h):
      pltpu.async_copy(sup_h.at[idx_v.at[p, 0]], rows_v.at[p], gsem.at[p])

    def wait_gather(p, sup_h):
      pltpu.make_async_copy(sup_h.at[idx_v.at[p, 0]], rows_v.at[p],
                            gsem.at[p]).wait()

    def fire_scatter(p):
      pltpu.async_copy(rows_v.at[p], acc.at[idx_v.at[p, 1]], ssem.at[p],
                       add=True)

    def wait_scatter(p):
      pltpu.make_async_copy(rows_v.at[p], acc.at[idx_v.at[p, 1]],
                            ssem.at[p]).wait()

    def run(sup_h):
      def body(j, carry):
        @pl.when(j >= _NB)
        def _():
          wait_scatter(j % _NB)

        fire_idx(j, j % _NB)

        @pl.when(j >= _LAG_G)
        def _():
          wait_idx((j - _LAG_G) % _NB)
          fire_gather((j - _LAG_G) % _NB, sup_h)

        @pl.when(j >= _LAG_S)
        def _():
          wait_gather((j - _LAG_S) % _NB, sup_h)
          fire_scatter((j - _LAG_S) % _NB)

        return carry

      lax.fori_loop(0, NCHT, body, 0)

      def tail_g(t, carry):
        wait_idx(t % _NB)
        fire_gather(t % _NB, sup_h)
        return carry

      def tail_s(t, carry):
        wait_gather(t % _NB, sup_h)
        fire_scatter(t % _NB)
        return carry

      def tail_w(t, carry):
        wait_scatter(t % _NB)
        return carry

      lax.fori_loop(NCHT - _LAG_G, NCHT, tail_g, 0)
      lax.fori_loop(NCHT - _LAG_S, NCHT, tail_s, 0)
      lax.fori_loop(NCHT - _NB, NCHT, tail_w, 0)

    @pl.when(cid == 0)
    def _():
      run(sup0_h)

    @pl.when(cid == 1)
    def _():
      run(sup1_h)

    plsc.subcore_barrier()
    rem = n_out - (_NUM_SUBCORES - 1) * RP

    @pl.when(sid < _NUM_SUBCORES - 1)
    def _():
      pltpu.sync_copy(acc.at[pl.ds(r0, RP)],
                      out_h.at[pl.ds(r0, RP), pl.ds(cid * Dc, Dc)])

    @pl.when(sid == _NUM_SUBCORES - 1)
    def _():
      r1 = (_NUM_SUBCORES - 1) * RP
      pltpu.sync_copy(acc.at[pl.ds(r1, rem)],
                      out_h.at[pl.ds(r1, rem), pl.ds(cid * Dc, Dc)])

  return agg(sup0, sup1, adj, zeros)


def kernel(fea, adj, W1, b1, W2, b2):
  N = fea.shape[0]
  adj32 = adj.astype(jnp.int32)
  s10, s11 = _mm_bias_split(fea, W1, b1, 200)
  x1 = _aggregate(s10, s11, adj32, _NACC)
  s20, s21 = _mm_bias_split(x1, W2, b2, 256)
  return _aggregate(s20, s21, adj32, N)


# speedup vs baseline: 1.1802x; 1.1802x over previous
import functools

import jax
import jax.numpy as jnp
from jax import lax
from jax.experimental import pallas as pl
from jax.experimental.pallas import tpu as pltpu
from jax.experimental.pallas import tpu_sc as plsc

_NUM_CORES = 2
_NUM_SUBCORES = 16

_C = 128
_NB = 10
_LAG_G = 1
_LAG_S = 5
_NACC = 10240


def _mm_bias_split(x, W, b, BR):
  N, K = x.shape
  Do = W.shape[1]
  Dc = Do // 2
  G = N // BR

  def body(x_ref, w_ref, b_ref, o0_ref, o1_ref):
    r = jnp.dot(x_ref[...], w_ref[...],
                preferred_element_type=jnp.float32) + b_ref[...]
    o0_ref[...] = r[:, :Dc]
    o1_ref[...] = r[:, Dc:]

  return pl.pallas_call(
      body,
      grid=(G,),
      in_specs=[
          pl.BlockSpec((BR, K), lambda i: (i, 0)),
          pl.BlockSpec((K, Do), lambda i: (0, 0)),
          pl.BlockSpec((1, Do), lambda i: (0, 0)),
      ],
      out_specs=[
          pl.BlockSpec((BR, Dc), lambda i: (i, 0)),
          pl.BlockSpec((BR, Dc), lambda i: (i, 0)),
      ],
      out_shape=[
          jax.ShapeDtypeStruct((N, Dc), jnp.float32),
          jax.ShapeDtypeStruct((N, Dc), jnp.float32),
      ],
  )(x, W, b.reshape(1, Do))


def _aggregate(sup0, sup1, adj, n_out):
  Dc = sup0.shape[1]
  E = adj.shape[1]
  NCG = E // _C
  NCH = NCG // _NUM_SUBCORES
  NEXTRA = NCG - NCH * _NUM_SUBCORES
  RP = _NACC // _NUM_SUBCORES

  zeros = jnp.zeros((RP, Dc), jnp.float32)
  mesh = plsc.VectorSubcoreMesh(core_axis_name="c", subcore_axis_name="s")

  @functools.partial(
      pl.kernel,
      mesh=mesh,
      compiler_params=pltpu.CompilerParams(use_tc_tiling_on_sc=False),
      out_type=jax.ShapeDtypeStruct((n_out, 2 * Dc), jnp.float32),
      scratch_types=[
          pltpu.VMEM((_NB, 2, _C), jnp.int32),
          pltpu.VMEM((_NB, _C, Dc), jnp.float32),
          pltpu.VMEM_SHARED((_NACC, Dc), jnp.float32),
          pltpu.SemaphoreType.DMA((_NB,)),
          pltpu.SemaphoreType.DMA((_NB,)),
          pltpu.SemaphoreType.DMA((_NB,)),
      ],
  )
  def agg(sup0_h, sup1_h, adj_h, zero_h, out_h, idx_v, rows_v, acc,
          isem, gsem, ssem):
    cid = lax.axis_index("c")
    sid = lax.axis_index("s")
    r0 = sid * RP
    NCHT = NCH + jnp.where(sid < NEXTRA, 1, 0)
    pltpu.sync_copy(zero_h, acc.at[pl.ds(r0, RP)])
    plsc.subcore_barrier()

    def fire_idx(j, p):
      e0 = (sid + _NUM_SUBCORES * j) * _C
      pltpu.async_copy(adj_h.at[0, pl.ds(e0, _C)], idx_v.at[p, 0], isem.at[p])
      pltpu.async_copy(adj_h.at[1, pl.ds(e0, _C)], idx_v.at[p, 1], isem.at[p])

    def wait_idx(p):
      pltpu.make_async_copy(adj_h.at[pl.ds(0, 2), pl.ds(0, _C)],
                            idx_v.at[p], isem.at[p]).wait()

    def fire_gather(p, sup_h):
      pltpu.async_copy(sup_h.at[idx_v.at[p, 0]], rows_v.at[p], gsem.at[p])

    def wait_gather(p, sup_h):
      pltpu.make_async_copy(sup_h.at[idx_v.at[p, 0]], rows_v.at[p],
                            gsem.at[p]).wait()

    def fire_scatter(p):
      pltpu.async_copy(rows_v.at[p], acc.at[idx_v.at[p, 1]], ssem.at[p],
                       add=True)

    def wait_scatter(p):
      pltpu.make_async_copy(rows_v.at[p], acc.at[idx_v.at[p, 1]],
                            ssem.at[p]).wait()

    def run(sup_h):
      def body(j, carry):
        @pl.when(j >= _NB)
        def _():
          wait_scatter(j % _NB)

        fire_idx(j, j % _NB)

        @pl.when(j >= _LAG_G)
        def _():
          wait_idx((j - _LAG_G) % _NB)
          fire_gather((j - _LAG_G) % _NB, sup_h)

        @pl.when(j >= _LAG_S)
        def _():
          wait_gather((j - _LAG_S) % _NB, sup_h)
          fire_scatter((j - _LAG_S) % _NB)

        return carry

      lax.fori_loop(0, NCHT, body, 0)

      def tail_g(t, carry):
        wait_idx(t % _NB)
        fire_gather(t % _NB, sup_h)
        return carry

      def tail_s(t, carry):
        wait_gather(t % _NB, sup_h)
        fire_scatter(t % _NB)
        return carry

      def tail_w(t, carry):
        wait_scatter(t % _NB)
        return carry

      lax.fori_loop(NCHT - _LAG_G, NCHT, tail_g, 0)
      lax.fori_loop(NCHT - _LAG_S, NCHT, tail_s, 0)
      lax.fori_loop(NCHT - _NB, NCHT, tail_w, 0)

    @pl.when(cid == 0)
    def _():
      run(sup0_h)

    @pl.when(cid == 1)
    def _():
      run(sup1_h)

    plsc.subcore_barrier()
    rem = n_out - (_NUM_SUBCORES - 1) * RP

    @pl.when(sid < _NUM_SUBCORES - 1)
    def _():
      pltpu.sync_copy(acc.at[pl.ds(r0, RP)],
                      out_h.at[pl.ds(r0, RP), pl.ds(cid * Dc, Dc)])

    @pl.when(sid == _NUM_SUBCORES - 1)
    def _():
      r1 = (_NUM_SUBCORES - 1) * RP
      pltpu.sync_copy(acc.at[pl.ds(r1, rem)],
                      out_h.at[pl.ds(r1, rem), pl.ds(cid * Dc, Dc)])

  return agg(sup0, sup1, adj, zeros)


def kernel(fea, adj, W1, b1, W2, b2):
  N = fea.shape[0]
  adj32 = adj.astype(jnp.int32)
  s10, s11 = _mm_bias_split(fea, W1, b1, 2000)
  x1 = _aggregate(s10, s11, adj32, _NACC)
  s20, s21 = _mm_bias_split(x1, W2, b2, 1024)
  return _aggregate(s20, s21, adj32, N)


# speedup vs baseline: 1.2097x; 1.0250x over previous
import functools

import jax
import jax.numpy as jnp
from jax import lax
from jax.experimental import pallas as pl
from jax.experimental.pallas import tpu as pltpu
from jax.experimental.pallas import tpu_sc as plsc

_NUM_CORES = 2
_NUM_SUBCORES = 16

_C = 128
_NB = 10
_LAG_G = 1
_LAG_S = 5
_NACC = 10240


def _mm_bias_split(x, W, b, BR):
  N, K = x.shape
  Do = W.shape[1]
  Dc = Do // 2
  G = N // BR

  def body(x_ref, w_ref, b_ref, o0_ref, o1_ref):
    r = jnp.dot(x_ref[...], w_ref[...],
                preferred_element_type=jnp.float32) + b_ref[...]
    o0_ref[...] = r[:, :Dc]
    o1_ref[...] = r[:, Dc:]

  return pl.pallas_call(
      body,
      grid=(G,),
      in_specs=[
          pl.BlockSpec((BR, K), lambda i: (i, 0)),
          pl.BlockSpec((K, Do), lambda i: (0, 0)),
          pl.BlockSpec((1, Do), lambda i: (0, 0)),
      ],
      out_specs=[
          pl.BlockSpec((BR, Dc), lambda i: (i, 0)),
          pl.BlockSpec((BR, Dc), lambda i: (i, 0)),
      ],
      out_shape=[
          jax.ShapeDtypeStruct((N, Dc), jnp.float32),
          jax.ShapeDtypeStruct((N, Dc), jnp.float32),
      ],
  )(x, W, b.reshape(1, Do))


def _aggregate(sup0, sup1, adj, n_out):
  Dc = sup0.shape[1]
  E = adj.shape[1]
  NCG = E // _C
  NCH = NCG // _NUM_SUBCORES
  NEXTRA = NCG - NCH * _NUM_SUBCORES
  RP = _NACC // _NUM_SUBCORES

  zeros = jnp.zeros((RP, Dc), jnp.float32)
  mesh = plsc.VectorSubcoreMesh(core_axis_name="c", subcore_axis_name="s")

  @functools.partial(
      pl.kernel,
      mesh=mesh,
      compiler_params=pltpu.CompilerParams(use_tc_tiling_on_sc=False),
      out_type=jax.ShapeDtypeStruct((n_out, 2 * Dc), jnp.float32),
      scratch_types=[
          pltpu.VMEM((_NB, 2, _C), jnp.int32),
          pltpu.VMEM((_NB, _C, Dc), jnp.float32),
          pltpu.VMEM_SHARED((_NACC, Dc), jnp.float32),
          pltpu.SemaphoreType.DMA((_NB,)),
          pltpu.SemaphoreType.DMA((_NB,)),
          pltpu.SemaphoreType.DMA((_NB,)),
      ],
  )
  def agg(sup0_h, sup1_h, adj_h, zero_h, out_h, idx_v, rows_v, acc,
          isem, gsem, ssem):
    cid = lax.axis_index("c")
    sid = lax.axis_index("s")
    r0 = sid * RP
    NCHT = NCH + jnp.where(sid < NEXTRA, 1, 0)
    pltpu.sync_copy(zero_h, acc.at[pl.ds(r0, RP)])
    plsc.subcore_barrier()

    def fire_idx(j, p):
      e0 = (sid + _NUM_SUBCORES * j) * _C
      pltpu.async_copy(adj_h.at[0, pl.ds(e0, _C)], idx_v.at[p, 0], isem.at[p])
      pltpu.async_copy(adj_h.at[1, pl.ds(e0, _C)], idx_v.at[p, 1], isem.at[p])

    def wait_idx(p):
      pltpu.make_async_copy(adj_h.at[pl.ds(0, 2), pl.ds(0, _C)],
                            idx_v.at[p], isem.at[p]).wait()

    def fire_gather(p, sup_h):
      pltpu.async_copy(sup_h.at[idx_v.at[p, 0]], rows_v.at[p], gsem.at[p])

    def wait_gather(p, sup_h):
      pltpu.make_async_copy(sup_h.at[idx_v.at[p, 0]], rows_v.at[p],
                            gsem.at[p]).wait()

    def fire_scatter(p):
      pltpu.async_copy(rows_v.at[p], acc.at[idx_v.at[p, 1]], ssem.at[p],
                       add=True)

    def wait_scatter(p):
      pltpu.make_async_copy(rows_v.at[p], acc.at[idx_v.at[p, 1]],
                            ssem.at[p]).wait()

    def run(sup_h):
      def body(j, carry):
        @pl.when(j >= _NB)
        def _():
          wait_scatter(j % _NB)

        fire_idx(j, j % _NB)

        @pl.when(j >= _LAG_G)
        def _():
          wait_idx((j - _LAG_G) % _NB)
          fire_gather((j - _LAG_G) % _NB, sup_h)

        @pl.when(j >= _LAG_S)
        def _():
          wait_gather((j - _LAG_S) % _NB, sup_h)
          fire_scatter((j - _LAG_S) % _NB)

        return carry

      lax.fori_loop(0, NCHT, body, 0)

      def tail_g(t, carry):
        wait_idx(t % _NB)
        fire_gather(t % _NB, sup_h)
        return carry

      def tail_s(t, carry):
        wait_gather(t % _NB, sup_h)
        fire_scatter(t % _NB)
        return carry

      def tail_w(t, carry):
        wait_scatter(t % _NB)
        return carry

      lax.fori_loop(NCHT - _LAG_G, NCHT, tail_g, 0)
      lax.fori_loop(NCHT - _LAG_S, NCHT, tail_s, 0)
      lax.fori_loop(NCHT - _NB, NCHT, tail_w, 0)

    @pl.when(cid == 0)
    def _():
      run(sup0_h)

    @pl.when(cid == 1)
    def _():
      run(sup1_h)

    plsc.subcore_barrier()
    rem = n_out - (_NUM_SUBCORES - 1) * RP

    @pl.when(sid < _NUM_SUBCORES - 1)
    def _():
      pltpu.sync_copy(acc.at[pl.ds(r0, RP)],
                      out_h.at[pl.ds(r0, RP), pl.ds(cid * Dc, Dc)])

    @pl.when(sid == _NUM_SUBCORES - 1)
    def _():
      r1 = (_NUM_SUBCORES - 1) * RP
      pltpu.sync_copy(acc.at[pl.ds(r1, rem)],
                      out_h.at[pl.ds(r1, rem), pl.ds(cid * Dc, Dc)])

  return agg(sup0, sup1, adj, zeros)


def kernel(fea, adj, W1, b1, W2, b2):
  N = fea.shape[0]
  adj32 = adj.astype(jnp.int32)
  s10, s11 = _mm_bias_split(fea, W1, b1, 10000)
  x1 = _aggregate(s10, s11, adj32, _NACC)
  s20, s21 = _mm_bias_split(x1, W2, b2, 10240)
  return _aggregate(s20, s21, adj32, N)
